# baseline (device time: 90127 ns/iter reference)
import jax
import jax.numpy as jnp
from jax import lax
from jax.experimental import pallas as pl
from jax.experimental.pallas import tpu as pltpu

N_DEV = 4
SCALE = 0.08838834764831843


def kernel(x, Wq, Wo, K_ext, V_ext):
    x2 = x[0].astype(jnp.bfloat16)
    wq = Wq.astype(jnp.bfloat16)
    wo = Wo.astype(jnp.bfloat16)
    k3 = jnp.transpose(K_ext[0], (1, 0, 2)).astype(jnp.bfloat16)
    v3 = jnp.transpose(V_ext[0], (1, 0, 2)).astype(jnp.bfloat16)

    Sq, D = x2.shape
    H, Skv, Dh = k3.shape

    def body(x_ref, wq_ref, wo_ref, k_ref, v_ref, out_ref,
             comm_ref, send_sems, recv_sems):
        my_i = lax.axis_index("i")
        p1 = N_DEV - 1 - my_i
        p2 = my_i + 1 - 2 * (my_i % 2)

        barrier_sem = pltpu.get_barrier_semaphore()
        for p in (p1, p2):
            pl.semaphore_signal(barrier_sem, inc=1, device_id=(p,),
                                device_id_type=pl.DeviceIdType.MESH)
        pl.semaphore_wait(barrier_sem, 2)

        q = lax.dot(x_ref[...], wq_ref[...],
                    preferred_element_type=jnp.float32)
        q = (q * SCALE).astype(jnp.bfloat16)

        acc = jnp.zeros((Sq, D), jnp.float32)
        for h in range(H):
            qh = q[:, h * Dh:(h + 1) * Dh]
            s = lax.dot_general(qh, k_ref[h], (((1,), (1,)), ((), ())),
                                preferred_element_type=jnp.float32)
            m = jnp.max(s, axis=1, keepdims=True)
            p = jnp.exp(s - m)
            l = jnp.sum(p, axis=1, keepdims=True)
            o = lax.dot(p.astype(jnp.bfloat16), v_ref[h],
                        preferred_element_type=jnp.float32)
            o = (o / l).astype(jnp.bfloat16)
            acc = acc + lax.dot(o, wo_ref[h * Dh:(h + 1) * Dh, :],
                                preferred_element_type=jnp.float32)

        out_ref[...] = acc

        for stage, partner in enumerate((p1, p2)):
            rdma = pltpu.make_async_remote_copy(
                src_ref=out_ref,
                dst_ref=comm_ref.at[stage],
                send_sem=send_sems.at[stage],
                recv_sem=recv_sems.at[stage],
                device_id=(partner,),
                device_id_type=pl.DeviceIdType.MESH,
            )
            rdma.start()
            rdma.wait()
            out_ref[...] = out_ref[...] + comm_ref[stage]

    out = pl.pallas_call(
        body,
        out_shape=jax.ShapeDtypeStruct((Sq, D), jnp.float32),
        in_specs=[pl.BlockSpec(memory_space=pltpu.VMEM)] * 5,
        out_specs=pl.BlockSpec(memory_space=pltpu.VMEM),
        scratch_shapes=[
            pltpu.VMEM((2, Sq, D), jnp.float32),
            pltpu.SemaphoreType.DMA((2,)),
            pltpu.SemaphoreType.DMA((2,)),
        ],
        compiler_params=pltpu.CompilerParams(collective_id=0),
    )(x2, wq, wo, k3, v3)
    return out[None]


# device time: 66211 ns/iter; 1.3612x vs baseline; 1.3612x over previous
import jax
import jax.numpy as jnp
from jax import lax
from jax.experimental import pallas as pl
from jax.experimental.pallas import tpu as pltpu

N_DEV = 4
SCALE = 0.08838834764831843
NC = 4


def kernel(x, Wq, Wo, K_ext, V_ext):
    x2 = x[0]
    k2 = K_ext.reshape(2048, 1024)
    v2 = V_ext.reshape(2048, 1024)

    Sq, D = x2.shape
    Skv = k2.shape[0]
    H, Dh = 8, 128
    R = Sq // NC

    def body(x_ref, wq_ref, wo_ref, k_ref, v_ref, out_ref,
             part_ref, s1_ref, red_ref, s2_ref,
             s1_send, s1_recv, s2_send, s2_recv):
        my_i = lax.axis_index("i")
        p1 = N_DEV - 1 - my_i
        p2 = my_i + 1 - 2 * (my_i % 2)

        barrier_sem = pltpu.get_barrier_semaphore()
        for p in (p1, p2):
            pl.semaphore_signal(barrier_sem, inc=1, device_id=(p,),
                                device_id_type=pl.DeviceIdType.MESH)
        pl.semaphore_wait(barrier_sem, 2)

        xb = x_ref[...].astype(jnp.bfloat16)
        wqb = wq_ref[...].astype(jnp.bfloat16)
        wob = wo_ref[...].astype(jnp.bfloat16)
        kb = [k_ref[:, h * Dh:(h + 1) * Dh].astype(jnp.bfloat16)
              for h in range(H)]
        vb = [v_ref[:, h * Dh:(h + 1) * Dh].astype(jnp.bfloat16)
              for h in range(H)]

        s1_rdmas = [None] * NC
        s2_rdmas = [None] * NC

        def compute_chunk(c):
            q = lax.dot(xb[c * R:(c + 1) * R, :], wqb,
                        preferred_element_type=jnp.float32)
            q = (q * SCALE).astype(jnp.bfloat16)
            acc = jnp.zeros((R, D), jnp.float32)
            for h in range(H):
                s = lax.dot_general(
                    q[:, h * Dh:(h + 1) * Dh], kb[h],
                    (((1,), (1,)), ((), ())),
                    preferred_element_type=jnp.float32)
                m = jnp.max(s, axis=1, keepdims=True)
                p = jnp.exp(s - m)
                l = jnp.sum(p, axis=1, keepdims=True)
                o = lax.dot(p.astype(jnp.bfloat16), vb[h],
                            preferred_element_type=jnp.float32)
                o = (o / l).astype(jnp.bfloat16)
                acc = acc + lax.dot(o, wob[h * Dh:(h + 1) * Dh, :],
                                    preferred_element_type=jnp.float32)
            part_ref[c] = acc.astype(jnp.bfloat16)

        def start_s1(c):
            r = pltpu.make_async_remote_copy(
                src_ref=part_ref.at[c], dst_ref=s1_ref.at[c],
                send_sem=s1_send.at[c], recv_sem=s1_recv.at[c],
                device_id=(p1,), device_id_type=pl.DeviceIdType.MESH)
            r.start()
            s1_rdmas[c] = r

        def handle_s1(c):
            s1_rdmas[c].wait_recv()
            red_ref[c] = part_ref[c] + s1_ref[c]
            r = pltpu.make_async_remote_copy(
                src_ref=red_ref.at[c], dst_ref=s2_ref.at[c],
                send_sem=s2_send.at[c], recv_sem=s2_recv.at[c],
                device_id=(p2,), device_id_type=pl.DeviceIdType.MESH)
            r.start()
            s2_rdmas[c] = r

        def finish(c):
            s2_rdmas[c].wait_recv()
            out_ref[c * R:(c + 1) * R, :] = (
                red_ref[c].astype(jnp.float32)
                + s2_ref[c].astype(jnp.float32))

        for c in range(NC):
            compute_chunk(c)
            start_s1(c)
            if c >= 1:
                handle_s1(c - 1)
            if c >= 2:
                finish(c - 2)
        handle_s1(NC - 1)
        finish(NC - 2)
        finish(NC - 1)

        for c in range(NC):
            s1_rdmas[c].wait_send()
            s2_rdmas[c].wait_send()

    out = pl.pallas_call(
        body,
        out_shape=jax.ShapeDtypeStruct((Sq, D), jnp.float32),
        in_specs=[pl.BlockSpec(memory_space=pltpu.VMEM)] * 5,
        out_specs=pl.BlockSpec(memory_space=pltpu.VMEM),
        scratch_shapes=[
            pltpu.VMEM((NC, R, D), jnp.bfloat16),
            pltpu.VMEM((NC, R, D), jnp.bfloat16),
            pltpu.VMEM((NC, R, D), jnp.bfloat16),
            pltpu.VMEM((NC, R, D), jnp.bfloat16),
            pltpu.SemaphoreType.DMA((NC,)),
            pltpu.SemaphoreType.DMA((NC,)),
            pltpu.SemaphoreType.DMA((NC,)),
            pltpu.SemaphoreType.DMA((NC,)),
        ],
        compiler_params=pltpu.CompilerParams(collective_id=0),
    )(x2, Wq, Wo, k2, v2)
    return out[None]


# device time: 59482 ns/iter; 1.5152x vs baseline; 1.1131x over previous
import jax
import jax.numpy as jnp
from jax import lax
from jax.experimental import pallas as pl
from jax.experimental.pallas import tpu as pltpu

N_DEV = 4
SCALE = 0.08838834764831843
NC = 4


def kernel(x, Wq, Wo, K_ext, V_ext):
    x2 = x[0]
    k2 = K_ext.reshape(2048, 1024).astype(jnp.bfloat16)
    v2 = V_ext.reshape(2048, 1024).astype(jnp.bfloat16)

    Sq, D = x2.shape
    Skv = k2.shape[0]
    H, Dh = 8, 128
    R = Sq // NC

    def body(x_ref, wq_ref, wo_ref, k_ref, v_ref, out_ref,
             part_ref, s1_ref, red_ref, s2_ref,
             s1_send, s1_recv, s2_send, s2_recv):
        my_i = lax.axis_index("i")
        p1 = N_DEV - 1 - my_i
        p2 = my_i + 1 - 2 * (my_i % 2)

        barrier_sem = pltpu.get_barrier_semaphore()
        for p in (p1, p2):
            pl.semaphore_signal(barrier_sem, inc=1, device_id=(p,),
                                device_id_type=pl.DeviceIdType.MESH)
        pl.semaphore_wait(barrier_sem, 2)

        xb = x_ref[...].astype(jnp.bfloat16)
        wqb = wq_ref[...].astype(jnp.bfloat16)
        wob = wo_ref[...].astype(jnp.bfloat16)
        kb = [k_ref[:, h * Dh:(h + 1) * Dh] for h in range(H)]
        vb = [v_ref[:, h * Dh:(h + 1) * Dh] for h in range(H)]

        s1_rdmas = [None] * NC
        s2_rdmas = [None] * NC

        def compute_chunk(c):
            q = lax.dot(xb[c * R:(c + 1) * R, :], wqb,
                        preferred_element_type=jnp.float32)
            q = (q * SCALE).astype(jnp.bfloat16)
            acc = jnp.zeros((R, D), jnp.float32)
            for h in range(H):
                s = lax.dot_general(
                    q[:, h * Dh:(h + 1) * Dh], kb[h],
                    (((1,), (1,)), ((), ())),
                    preferred_element_type=jnp.float32)
                m = jnp.max(s, axis=1, keepdims=True)
                p = jnp.exp(s - m)
                l = jnp.sum(p, axis=1, keepdims=True)
                o = lax.dot(p.astype(jnp.bfloat16), vb[h],
                            preferred_element_type=jnp.float32)
                o = (o / l).astype(jnp.bfloat16)
                acc = acc + lax.dot(o, wob[h * Dh:(h + 1) * Dh, :],
                                    preferred_element_type=jnp.float32)
            part_ref[c] = acc.astype(jnp.bfloat16)

        def start_s1(c):
            r = pltpu.make_async_remote_copy(
                src_ref=part_ref.at[c], dst_ref=s1_ref.at[c],
                send_sem=s1_send.at[c], recv_sem=s1_recv.at[c],
                device_id=(p1,), device_id_type=pl.DeviceIdType.MESH)
            r.start()
            s1_rdmas[c] = r

        def handle_s1(c):
            s1_rdmas[c].wait_recv()
            red_ref[c] = part_ref[c] + s1_ref[c]
            r = pltpu.make_async_remote_copy(
                src_ref=red_ref.at[c], dst_ref=s2_ref.at[c],
                send_sem=s2_send.at[c], recv_sem=s2_recv.at[c],
                device_id=(p2,), device_id_type=pl.DeviceIdType.MESH)
            r.start()
            s2_rdmas[c] = r

        def finish(c):
            s2_rdmas[c].wait_recv()
            out_ref[c * R:(c + 1) * R, :] = (
                red_ref[c].astype(jnp.float32)
                + s2_ref[c].astype(jnp.float32))

        for c in range(NC):
            compute_chunk(c)
            start_s1(c)
            if c >= 1:
                handle_s1(c - 1)
            if c >= 2:
                finish(c - 2)
        handle_s1(NC - 1)
        finish(NC - 2)
        finish(NC - 1)

        for c in range(NC):
            s1_rdmas[c].wait_send()
            s2_rdmas[c].wait_send()

    out = pl.pallas_call(
        body,
        out_shape=jax.ShapeDtypeStruct((Sq, D), jnp.float32),
        in_specs=[pl.BlockSpec(memory_space=pltpu.VMEM)] * 5,
        out_specs=pl.BlockSpec(memory_space=pltpu.VMEM),
        scratch_shapes=[
            pltpu.VMEM((NC, R, D), jnp.bfloat16),
            pltpu.VMEM((NC, R, D), jnp.bfloat16),
            pltpu.VMEM((NC, R, D), jnp.bfloat16),
            pltpu.VMEM((NC, R, D), jnp.bfloat16),
            pltpu.SemaphoreType.DMA((NC,)),
            pltpu.SemaphoreType.DMA((NC,)),
            pltpu.SemaphoreType.DMA((NC,)),
            pltpu.SemaphoreType.DMA((NC,)),
        ],
        compiler_params=pltpu.CompilerParams(collective_id=0),
    )(x2, Wq, Wo, k2, v2)
    return out[None]
